# baseline (speedup 1.0000x reference)
import functools

import jax
import jax.numpy as jnp
from jax import lax
from jax.experimental import pallas as pl
from jax.experimental.pallas import tpu as pltpu
from jax.experimental.pallas import tpu_sc as plsc

VOCAB = 100000
EMBED_DIM = 50
PAD_DIM = 64
BATCH = 4096
SEQ = 50

_B = BATCH * SEQ
_NC, _NS = 2, 16
_NW = _NC * _NS
_CHUNK = 128
_PER_W = _B // _NW
_NCHUNK = _PER_W // _CHUNK


def _repack_rows(raw, packed):
    def body(r, _):
        for c in (0, 16, 32):
            packed[r, pl.ds(c, 16)] = raw[r, pl.ds(c, 16)]
        packed[r, pl.ds(EMBED_DIM - 16, 16)] = raw[r, pl.ds(EMBED_DIM - 16, 16)]
        return _

    lax.fori_loop(0, _CHUNK, body, 0)


def _gather_body(
    table_hbm, idx_hbm, out_hbm,
    idx_v, raw_a, raw_b, pk_a, pk_b, gsem_a, gsem_b, osem_a, osem_b,
):
    wid = lax.axis_index("c") * _NS + lax.axis_index("s")
    row_base = wid * _PER_W
    pltpu.sync_copy(idx_hbm.at[pl.ds(row_base, _PER_W)], idx_v)

    raws = (raw_a, raw_b)
    pks = (pk_a, pk_b)
    gsems = (gsem_a, gsem_b)
    osems = (osem_a, osem_b)

    pltpu.async_copy(table_hbm.at[idx_v.at[pl.ds(0, _CHUNK)]], raws[0], gsems[0])
    for j in range(_NCHUNK):
        nxt = j + 1
        if nxt < _NCHUNK:
            pltpu.async_copy(
                table_hbm.at[idx_v.at[pl.ds(nxt * _CHUNK, _CHUNK)]],
                raws[nxt % 2],
                gsems[nxt % 2],
            )
        pltpu.make_async_copy(
            table_hbm.at[idx_v.at[pl.ds(j * _CHUNK, _CHUNK)]],
            raws[j % 2],
            gsems[j % 2],
        ).wait()
        if j >= 2:
            pltpu.make_async_copy(
                pks[j % 2],
                out_hbm.at[pl.ds(row_base + (j - 2) * _CHUNK, _CHUNK)],
                osems[j % 2],
            ).wait()
        _repack_rows(raws[j % 2], pks[j % 2])
        pltpu.async_copy(
            pks[j % 2],
            out_hbm.at[pl.ds(row_base + j * _CHUNK, _CHUNK)],
            osems[j % 2],
        )
    for j in (_NCHUNK - 2, _NCHUNK - 1):
        pltpu.make_async_copy(
            pks[j % 2],
            out_hbm.at[pl.ds(row_base + j * _CHUNK, _CHUNK)],
            osems[j % 2],
        ).wait()


@jax.jit
def _embed_gather(table_padded, idx_flat):
    k = functools.partial(
        pl.kernel,
        out_type=jax.ShapeDtypeStruct((_B, EMBED_DIM), jnp.float32),
        mesh=plsc.VectorSubcoreMesh(core_axis_name="c", subcore_axis_name="s"),
        scratch_types=[
            pltpu.VMEM((_PER_W,), jnp.int32),
            pltpu.VMEM((_CHUNK, PAD_DIM), jnp.float32),
            pltpu.VMEM((_CHUNK, PAD_DIM), jnp.float32),
            pltpu.VMEM((_CHUNK, EMBED_DIM), jnp.float32),
            pltpu.VMEM((_CHUNK, EMBED_DIM), jnp.float32),
            pltpu.SemaphoreType.DMA,
            pltpu.SemaphoreType.DMA,
            pltpu.SemaphoreType.DMA,
            pltpu.SemaphoreType.DMA,
        ],
        compiler_params=pltpu.CompilerParams(use_tc_tiling_on_sc=False),
    )(_gather_body)
    return k(table_padded, idx_flat)


def kernel(indices, table):
    table_padded = jnp.pad(table, ((0, 0), (0, PAD_DIM - EMBED_DIM)))
    idx_flat = indices.reshape(_B)
    out = _embed_gather(table_padded, idx_flat)
    return out.reshape(BATCH, SEQ, EMBED_DIM)


# speedup vs baseline: 1.2967x; 1.2967x over previous
import functools

import jax
import jax.numpy as jnp
from jax import lax
from jax.experimental import pallas as pl
from jax.experimental.pallas import tpu as pltpu
from jax.experimental.pallas import tpu_sc as plsc

VOCAB = 100000
EMBED_DIM = 50
PAD_DIM = 64
BATCH = 4096
SEQ = 50

_B = BATCH * SEQ
_NC, _NS = 2, 16
_NW = _NC * _NS
_BPW = BATCH // _NW
_PER_W = _B // _NW
_CHUNK = 100
_SUPER = 4
_NSUPER = _BPW // _SUPER


def _repack_half(raw, pk, half):
    def body(s, _):
        for b, roff in ((2 * half, 0), (2 * half + 1, EMBED_DIM)):
            r = s + roff
            for c in (0, 16, 32):
                pk[b, s, pl.ds(c, 16)] = raw[r, pl.ds(c, 16)]
            pk[b, s, pl.ds(EMBED_DIM - 16, 16)] = raw[r, pl.ds(EMBED_DIM - 16, 16)]
        return _

    lax.fori_loop(0, SEQ, body, 0)


def _gather_body(
    table_hbm, idx_hbm, out_hbm,
    idx_v, stage_a, stage_b, raw_a, raw_b, pk_a, pk_b,
    gsem_a, gsem_b, osem_a, osem_b,
):
    wid = lax.axis_index("c") * _NS + lax.axis_index("s")
    base = wid * _PER_W
    pltpu.sync_copy(idx_hbm.at[pl.ds(base, _PER_W)], idx_v.at[pl.ds(0, _PER_W)])

    ivec = lax.iota(jnp.int32, 16)
    stages = (stage_a, stage_b)
    raws = (raw_a, raw_b)
    pks = (pk_a, pk_b)
    gsems = (gsem_a, gsem_b)
    osems = (osem_a, osem_b)

    def stage_idx(ch):
        st = stages[ch % 2]
        off = ch * _CHUNK
        for t in range(7):
            st[pl.ds(t * 16, 16)] = plsc.load_gather(idx_v, [ivec + (off + t * 16)])
        return st

    def fire(ch):
        st = stage_idx(ch)
        pltpu.async_copy(
            table_hbm.at[st.at[pl.ds(0, _CHUNK)]], raws[ch % 2], gsems[ch % 2]
        )

    def wait_gather(ch):
        pltpu.make_async_copy(
            table_hbm.at[stages[ch % 2].at[pl.ds(0, _CHUNK)]],
            raws[ch % 2],
            gsems[ch % 2],
        ).wait()

    def out_copy(k):
        return (
            pks[k % 2],
            out_hbm.at[pl.ds(wid * _BPW + k * _SUPER, _SUPER)],
            osems[k % 2],
        )

    fire(0)
    for k in range(_NSUPER):
        fire(2 * k + 1)
        wait_gather(2 * k)
        if k >= 2:
            src, dst, sem = out_copy(k - 2)
            pltpu.make_async_copy(src, dst, sem).wait()
        _repack_half(raws[(2 * k) % 2], pks[k % 2], 0)
        if k + 1 < _NSUPER:
            fire(2 * k + 2)
        wait_gather(2 * k + 1)
        _repack_half(raws[(2 * k + 1) % 2], pks[k % 2], 1)
        src, dst, sem = out_copy(k)
        pltpu.async_copy(src, dst, sem)
    for k in (_NSUPER - 2, _NSUPER - 1):
        src, dst, sem = out_copy(k)
        pltpu.make_async_copy(src, dst, sem).wait()


@jax.jit
def _embed_gather(table_padded, idx_flat):
    k = functools.partial(
        pl.kernel,
        out_type=jax.ShapeDtypeStruct((BATCH, SEQ, EMBED_DIM), jnp.float32),
        mesh=plsc.VectorSubcoreMesh(core_axis_name="c", subcore_axis_name="s"),
        scratch_types=[
            pltpu.VMEM((_PER_W + 16,), jnp.int32),
            pltpu.VMEM((112,), jnp.int32),
            pltpu.VMEM((112,), jnp.int32),
            pltpu.VMEM((_CHUNK, PAD_DIM), jnp.float32),
            pltpu.VMEM((_CHUNK, PAD_DIM), jnp.float32),
            pltpu.VMEM((_SUPER, SEQ, EMBED_DIM), jnp.float32),
            pltpu.VMEM((_SUPER, SEQ, EMBED_DIM), jnp.float32),
            pltpu.SemaphoreType.DMA,
            pltpu.SemaphoreType.DMA,
            pltpu.SemaphoreType.DMA,
            pltpu.SemaphoreType.DMA,
        ],
        compiler_params=pltpu.CompilerParams(
            use_tc_tiling_on_sc=False, needs_layout_passes=False
        ),
    )(_gather_body)
    return k(table_padded, idx_flat)


def kernel(indices, table):
    table_padded = jnp.pad(table, ((0, 0), (0, PAD_DIM - EMBED_DIM)))
    idx_flat = indices.reshape(_B)
    return _embed_gather(table_padded, idx_flat)
